# Initial kernel scaffold; baseline (speedup 1.0000x reference)
import functools
import jax
import jax.numpy as jnp
from jax import lax
from jax.experimental import pallas as pl
from jax.experimental.pallas import tpu as pltpu
from jax.experimental.pallas import tpu_sc as plsc

N = 10000
E = 160000
D = 256
K = 2000

NPAD = 10240
NC = 2
NS = 16
NW = NC * NS
EPW = E // NW
EPW_PAD = 5008
CHUNK = NPAD // NS


def _mesh():
    return plsc.VectorSubcoreMesh(core_axis_name="c", subcore_axis_name="s",
                                  num_cores=NC, num_subcores=NS)


def _matvec_body(x_ref, w_ref, o_ref):
    o_ref[...] = jnp.sum(x_ref[...] * w_ref[...], axis=1, keepdims=True)


def _matvec(x, w_row):
    return pl.pallas_call(
        _matvec_body,
        grid=(10,),
        in_specs=[
            pl.BlockSpec((1000, D), lambda i: (i, 0)),
            pl.BlockSpec((1, D), lambda i: (0, 0)),
        ],
        out_specs=pl.BlockSpec((1000, 1), lambda i: (i, 0)),
        out_shape=jax.ShapeDtypeStruct((N, 1), jnp.float32),
    )(x, w_row)


def _segsum_kernel(pre_hbm, dst_hbm, src_hbm, val_hbm, part_hbm,
                   presup_v, dst_v, val_v, msg_v, src_v, zero_v, acc_sh):
    c = lax.axis_index("c")
    s = lax.axis_index("s")
    w = s * NC + c

    for j in range(40):
        zero_v[pl.ds(j * 16, 16)] = jnp.zeros((16,), jnp.float32)

    @pl.when(s < NS - 1)
    def _():
        pltpu.sync_copy(zero_v, acc_sh.at[pl.ds(s * 640, 640)])

    @pl.when(s == NS - 1)
    def _():
        pltpu.sync_copy(zero_v.at[pl.ds(0, 400)], acc_sh.at[pl.ds(9600, 400)])

    pltpu.sync_copy(pre_hbm, presup_v)
    base = w * EPW
    pltpu.sync_copy(dst_hbm.at[pl.ds(base, EPW)], dst_v.at[pl.ds(0, EPW)])
    pltpu.sync_copy(val_hbm.at[pl.ds(base, EPW)], val_v.at[pl.ds(0, EPW)])
    pltpu.sync_copy(src_hbm.at[pl.ds(base, EPW)], src_v)

    plsc.subcore_barrier()

    def body(i, carry):
        sl = pl.ds(i * 16, 16)
        d = dst_v[sl]
        d = jnp.minimum(jnp.maximum(d, 0), N - 1)
        p = plsc.load_gather(presup_v, [d])
        msg_v[sl] = p * val_v[sl]
        return carry

    lax.fori_loop(0, EPW_PAD // 16, body, 0)

    pltpu.sync_copy(msg_v.at[pl.ds(0, EPW)], acc_sh.at[src_v], add=True)

    plsc.subcore_barrier()

    @pl.when(s < NS - 1)
    def _():
        pltpu.sync_copy(acc_sh.at[pl.ds(s * 640, 640)],
                        part_hbm.at[c, pl.ds(s * 640, 640)])

    @pl.when(s == NS - 1)
    def _():
        pltpu.sync_copy(acc_sh.at[pl.ds(9600, 400)],
                        part_hbm.at[c, pl.ds(9600, 400)])


def _segsum(pre, dst, src, val):
    return pl.kernel(
        _segsum_kernel,
        out_type=jax.ShapeDtypeStruct((NC, N), jnp.float32),
        mesh=_mesh(),
        scratch_types=[
            pltpu.VMEM((N,), jnp.float32),
            pltpu.VMEM((EPW_PAD,), jnp.int32),
            pltpu.VMEM((EPW_PAD,), jnp.float32),
            pltpu.VMEM((EPW_PAD,), jnp.float32),
            pltpu.VMEM((EPW,), jnp.int32),
            pltpu.VMEM((640,), jnp.float32),
            pltpu.VMEM_SHARED((N,), jnp.float32),
        ],
    )(pre, dst, src, val)


def _keys_body(p_ref, b_ref, k_ref):
    i = pl.program_id(0)
    score = p_ref[0] + p_ref[1] + b_ref[0, 0]
    t = jnp.tanh(score)
    bits = lax.bitcast_convert_type(t, jnp.uint32)
    flip = jnp.where(bits >> 31 == 1,
                     jnp.uint32(0xFFFFFFFF), jnp.uint32(0x80000000))
    u = bits ^ flip
    kdesc = ~u
    row = jax.lax.broadcasted_iota(jnp.int32, (8, 128), 0)
    col = jax.lax.broadcasted_iota(jnp.int32, (8, 128), 1)
    gidx = (i * 8 + row) * 128 + col
    kdesc = jnp.where(gidx < N, kdesc, jnp.uint32(0xFFFFFFFF))
    k_ref[...] = lax.bitcast_convert_type(kdesc, jnp.int32)


def _keys(p, b2d):
    return pl.pallas_call(
        _keys_body,
        grid=(10,),
        in_specs=[
            pl.BlockSpec((2, 8, 128), lambda i: (0, i, 0)),
            pl.BlockSpec((1, 1), lambda i: (0, 0)),
        ],
        out_specs=pl.BlockSpec((8, 128), lambda i: (i, 0)),
        out_shape=jax.ShapeDtypeStruct((NPAD // 128, 128), jnp.int32),
    )(p, b2d)


def _gather_kernel(idx_hbm, val_hbm, x_hbm, out_hbm, idx_v, val_v, rows_v):
    c = lax.axis_index("c")
    s = lax.axis_index("s")
    w = s * NC + c

    def do(nrows):
        base = w * 64
        pltpu.sync_copy(idx_hbm.at[pl.ds(base, nrows)], idx_v.at[pl.ds(0, nrows)])
        pltpu.sync_copy(val_hbm.at[pl.ds(base, nrows)], val_v.at[pl.ds(0, nrows)])
        pltpu.sync_copy(x_hbm.at[idx_v.at[pl.ds(0, nrows)]],
                        rows_v.at[pl.ds(0, nrows)])
        for g in range(nrows // 16):
            vv = val_v[pl.ds(g * 16, 16)]
            for j in range(16):
                sv = jnp.full((16,), 0.0, jnp.float32) + lax.dynamic_index_in_dim(
                    vv, j, keepdims=False)
                r = g * 16 + j
                for cb in range(D // 16):
                    sl = pl.ds(cb * 16, 16)
                    rows_v[r, sl] = rows_v[r, sl] * sv
        pltpu.sync_copy(rows_v.at[pl.ds(0, nrows)],
                        out_hbm.at[pl.ds(base, nrows)])

    @pl.when(w < NW - 1)
    def _():
        do(64)

    @pl.when(w == NW - 1)
    def _():
        do(K - 64 * (NW - 1))


def _gather_scale(topidx, topval, x):
    return pl.kernel(
        _gather_kernel,
        out_type=jax.ShapeDtypeStruct((K, D), jnp.float32),
        mesh=_mesh(),
        scratch_types=[
            pltpu.VMEM((64,), jnp.int32),
            pltpu.VMEM((64,), jnp.float32),
            pltpu.VMEM((64, D), jnp.float32),
        ],
    )(topidx, topval, x)


def kernel(x, edge_index, edge_vals, W0, b):
    src = edge_index[0]
    dst = edge_index[1]
    w_row = W0.reshape(1, D)

    pre = _matvec(x, w_row).reshape(N)
    partials = _segsum(pre, dst, src, edge_vals)

    p_pad = jnp.pad(partials, ((0, 0), (0, NPAD - N))).reshape(2, NPAD // 128, 128)
    keys_i32 = _keys(p_pad, b.reshape(1, 1)).reshape(NPAD)

    ku = lax.bitcast_convert_type(keys_i32, jnp.uint32)
    order = jnp.argsort(ku, stable=True)
    topidx = order[:K].astype(jnp.int32)
    ksel = ku[order[:K]]
    uu = ~ksel
    bits = jnp.where(uu >= jnp.uint32(0x80000000),
                     uu ^ jnp.uint32(0x80000000), ~uu)
    topval = lax.bitcast_convert_type(bits, jnp.float32)

    out = _gather_scale(topidx, topval, x)
    return out


# speedup vs baseline: 11.5478x; 11.5478x over previous
import functools
import jax
import jax.numpy as jnp
from jax import lax
from jax.experimental import pallas as pl
from jax.experimental.pallas import tpu as pltpu
from jax.experimental.pallas import tpu_sc as plsc

N = 10000
E = 160000
D = 256
K = 2000

NPAD = 10240
KPAD = 2048
NC = 2
NS = 16
NW = NC * NS
EPW = 5120
EPADT = EPW * NW


def _mesh():
    return plsc.VectorSubcoreMesh(core_axis_name="c", subcore_axis_name="s",
                                  num_cores=NC, num_subcores=NS)


_SC_PARAMS = pltpu.CompilerParams(needs_layout_passes=False)


def _matvec_body(x_ref, w_ref, o_ref):
    o_ref[...] = jnp.dot(x_ref[...], w_ref[...])


def _matvec(x, w_col):
    return pl.pallas_call(
        _matvec_body,
        grid=(10,),
        in_specs=[
            pl.BlockSpec((1024, D), lambda i: (i, 0)),
            pl.BlockSpec((D, 1), lambda i: (0, 0)),
        ],
        out_specs=pl.BlockSpec((1024, 1), lambda i: (i, 0)),
        out_shape=jax.ShapeDtypeStruct((NPAD, 1), jnp.float32),
    )(x, w_col)


def _segsum_kernel(pre_hbm, dst_hbm, src_hbm, val_hbm, part_hbm,
                   presup_v, dst_v, val_v, msg_v, src_v, zero_v, acc_sh):
    c = lax.axis_index("c")
    s = lax.axis_index("s")
    w = s * NC + c

    for j in range(40):
        zero_v[pl.ds(j * 16, 16)] = jnp.zeros((16,), jnp.float32)
    pltpu.sync_copy(zero_v, acc_sh.at[pl.ds(s * 640, 640)])

    pltpu.sync_copy(pre_hbm, presup_v)
    base = w * EPW
    pltpu.sync_copy(dst_hbm.at[pl.ds(base, EPW)], dst_v)
    pltpu.sync_copy(val_hbm.at[pl.ds(base, EPW)], val_v)
    pltpu.sync_copy(src_hbm.at[pl.ds(base, EPW)], src_v)

    plsc.subcore_barrier()

    def body(i, carry):
        sl = pl.ds(i * 16, 16)
        d = dst_v[sl]
        p = plsc.load_gather(presup_v, [d])
        msg_v[sl] = p * val_v[sl]
        return carry

    lax.fori_loop(0, EPW // 16, body, 0)

    pltpu.sync_copy(msg_v, acc_sh.at[src_v], add=True)

    plsc.subcore_barrier()

    pltpu.sync_copy(acc_sh.at[pl.ds(s * 640, 640)],
                    part_hbm.at[c, pl.ds(s * 640, 640)])


def _segsum(pre, dst, src, val):
    return pl.kernel(
        _segsum_kernel,
        out_type=jax.ShapeDtypeStruct((NC, NPAD), jnp.float32),
        mesh=_mesh(),
        compiler_params=_SC_PARAMS,
        scratch_types=[
            pltpu.VMEM((NPAD,), jnp.float32),
            pltpu.VMEM((EPW,), jnp.int32),
            pltpu.VMEM((EPW,), jnp.float32),
            pltpu.VMEM((EPW,), jnp.float32),
            pltpu.VMEM((EPW,), jnp.int32),
            pltpu.VMEM((640,), jnp.float32),
            pltpu.VMEM_SHARED((NPAD,), jnp.float32),
        ],
    )(pre, dst, src, val)


def _keys_body(p_ref, b_ref, k_ref):
    i = pl.program_id(0)
    score = p_ref[0] + p_ref[1] + b_ref[0, 0]
    t = jnp.tanh(score)
    bits = lax.bitcast_convert_type(t, jnp.uint32)
    flip = jnp.where(bits >> 31 == 1,
                     jnp.uint32(0xFFFFFFFF), jnp.uint32(0x80000000))
    u = bits ^ flip
    kdesc = ~u
    row = jax.lax.broadcasted_iota(jnp.int32, (8, 128), 0)
    col = jax.lax.broadcasted_iota(jnp.int32, (8, 128), 1)
    gidx = (i * 8 + row) * 128 + col
    kdesc = jnp.where(gidx < N, kdesc, jnp.uint32(0xFFFFFFFF))
    k_ref[...] = lax.bitcast_convert_type(kdesc, jnp.int32)


def _keys(p, b2d):
    return pl.pallas_call(
        _keys_body,
        grid=(10,),
        in_specs=[
            pl.BlockSpec((2, 8, 128), lambda i: (0, i, 0)),
            pl.BlockSpec((1, 1), lambda i: (0, 0)),
        ],
        out_specs=pl.BlockSpec((8, 128), lambda i: (i, 0)),
        out_shape=jax.ShapeDtypeStruct((NPAD // 128, 128), jnp.int32),
    )(p, b2d)


CHUNK = NPAD // NS
NV = CHUNK // 16


def _dup_and_hist(dig):
    lanes = lax.iota(jnp.int32, 16)
    dupRank = jnp.zeros((16,), jnp.int32)
    hist = jnp.zeros((16,), jnp.int32)
    for base in (0, 6, 12):
        rel = dig - base
        inseg = jnp.logical_and(rel >= 0, rel < 6)
        relc = jnp.clip(rel, 0, 5) * 5
        contrib = jnp.where(inseg, jnp.left_shift(jnp.int32(1), relc), 0)
        pre = jnp.cumsum(contrib) - contrib
        dupRank = dupRank + jnp.where(
            inseg, jnp.right_shift(pre, relc) & 31, 0)
        tot = jnp.sum(contrib)
        hrel = jnp.clip(lanes - base, 0, 5) * 5
        hseg = jnp.where(jnp.logical_and(lanes >= base, lanes < base + 6),
                         jnp.right_shift(tot, hrel) & 31, 0)
        hist = hist + hseg
    return dupRank, hist


def _sort_kernel(keys_hbm, tidx_hbm, tval_hbm,
                 kv, vv, digc, dupc, histc, posb, cnt_v, g_v, pub_v, outv,
                 bufA_k, bufA_v, bufB_k, bufB_v, grid_sh):
    c = lax.axis_index("c")
    s = lax.axis_index("s")

    @pl.when(c == 0)
    def _():
        pltpu.sync_copy(keys_hbm.at[pl.ds(s * CHUNK, CHUNK)], kv)
        for j in range(NV):
            vv[pl.ds(j * 16, 16)] = s * CHUNK + j * 16 + lax.iota(jnp.int32, 16)

        for p in range(8):
            sh = 4 * p

            def ph1(g, histTile):
                sl = pl.ds(g * 16, 16)
                dig = lax.shift_right_logical(kv[sl], sh) & 15
                dr, h = _dup_and_hist(dig)
                digc[sl] = dig
                dupc[sl] = dr
                histc[sl] = h
                return histTile + h

            histTile = lax.fori_loop(0, NV, ph1, jnp.zeros((16,), jnp.int32))
            pub_v[...] = histTile
            pltpu.sync_copy(pub_v, grid_sh.at[pl.ds(s * 16, 16)])
            plsc.subcore_barrier()

            pltpu.sync_copy(grid_sh, g_v)
            colTot = jnp.zeros((16,), jnp.int32)
            preT = jnp.zeros((16,), jnp.int32)
            for j in range(NS):
                gj = g_v[pl.ds(j * 16, 16)]
                colTot = colTot + gj
                preT = preT + jnp.where(j < s, gj, 0)
            excl = jnp.cumsum(colTot) - colTot
            cnt_v[...] = excl + preT

            def ph3(g, carry):
                sl = pl.ds(g * 16, 16)
                dig = digc[sl]
                base = plsc.load_gather(cnt_v, [dig])
                posb[sl] = base + dupc[sl]
                cnt_v[...] = cnt_v[...] + histc[sl]
                return carry

            lax.fori_loop(0, NV, ph3, 0)

            dst_k = bufB_k if p % 2 == 0 else bufA_k
            dst_v = bufB_v if p % 2 == 0 else bufA_v
            pltpu.sync_copy(kv, dst_k.at[posb])
            pltpu.sync_copy(vv, dst_v.at[posb])
            plsc.subcore_barrier()
            pltpu.sync_copy(dst_k.at[pl.ds(s * CHUNK, CHUNK)], kv)
            pltpu.sync_copy(dst_v.at[pl.ds(s * CHUNK, CHUNK)], vv)

        @pl.when(s * CHUNK < KPAD)
        def _():
            nw = min(CHUNK, KPAD)
            base = s * CHUNK

            def emit(n):
                pltpu.sync_copy(vv.at[pl.ds(0, n)], tidx_hbm.at[pl.ds(base, n)])
                for j in range(n // 16):
                    sl = pl.ds(j * 16, 16)
                    u = ~kv[sl]
                    bits = jnp.where(u < 0, u ^ jnp.int32(-2147483648), ~u)
                    outv[sl] = lax.bitcast_convert_type(bits, jnp.float32)
                pltpu.sync_copy(outv.at[pl.ds(0, n)], tval_hbm.at[pl.ds(base, n)])

            if KPAD % CHUNK == 0:
                emit(nw)
            else:
                @pl.when(s < KPAD // CHUNK)
                def _():
                    emit(nw)

                @pl.when(s == KPAD // CHUNK)
                def _():
                    emit(KPAD % CHUNK)


def _sort_topk(keys):
    return pl.kernel(
        _sort_kernel,
        out_type=[jax.ShapeDtypeStruct((KPAD,), jnp.int32),
                  jax.ShapeDtypeStruct((KPAD,), jnp.float32)],
        mesh=_mesh(),
        compiler_params=_SC_PARAMS,
        scratch_types=[
            pltpu.VMEM((CHUNK,), jnp.int32),
            pltpu.VMEM((CHUNK,), jnp.int32),
            pltpu.VMEM((CHUNK,), jnp.int32),
            pltpu.VMEM((CHUNK,), jnp.int32),
            pltpu.VMEM((CHUNK,), jnp.int32),
            pltpu.VMEM((CHUNK,), jnp.int32),
            pltpu.VMEM((16,), jnp.int32),
            pltpu.VMEM((NS * 16,), jnp.int32),
            pltpu.VMEM((16,), jnp.int32),
            pltpu.VMEM((CHUNK,), jnp.float32),
            pltpu.VMEM_SHARED((NPAD,), jnp.int32),
            pltpu.VMEM_SHARED((NPAD,), jnp.int32),
            pltpu.VMEM_SHARED((NPAD,), jnp.int32),
            pltpu.VMEM_SHARED((NPAD,), jnp.int32),
            pltpu.VMEM_SHARED((NS * 16,), jnp.int32),
        ],
    )(keys)


def _gather_kernel(idx_hbm, val_hbm, x_hbm, out_hbm, idx_v, val_v, rows_v):
    c = lax.axis_index("c")
    s = lax.axis_index("s")
    w = s * NC + c

    @pl.when(w < KPAD // 128)
    def _():
        base = w * 128
        pltpu.sync_copy(idx_hbm.at[pl.ds(base, 128)], idx_v)
        pltpu.sync_copy(val_hbm.at[pl.ds(base, 128)], val_v)
        pltpu.sync_copy(x_hbm.at[idx_v], rows_v)
        lanes = lax.iota(jnp.int32, 16)
        for g in range(8):
            vv = val_v[pl.ds(g * 16, 16)]
            for j in range(16):
                sv = jnp.sum(jnp.where(lanes == j, vv, 0.0))
                r = g * 16 + j
                for cb in range(D // 16):
                    sl = pl.ds(cb * 16, 16)
                    rows_v[r, sl] = rows_v[r, sl] * sv
        pltpu.sync_copy(rows_v, out_hbm.at[pl.ds(base, 128)])


def _gather_scale(topidx, topval, x):
    return pl.kernel(
        _gather_kernel,
        out_type=jax.ShapeDtypeStruct((KPAD, D), jnp.float32),
        mesh=_mesh(),
        compiler_params=_SC_PARAMS,
        scratch_types=[
            pltpu.VMEM((128,), jnp.int32),
            pltpu.VMEM((128,), jnp.float32),
            pltpu.VMEM((128, D), jnp.float32),
        ],
    )(topidx, topval, x)


def kernel(x, edge_index, edge_vals, W0, b):
    src = edge_index[0]
    dst = edge_index[1]


    srcp = jnp.pad(src, (0, EPADT - E))
    dstp = jnp.pad(dst, (0, EPADT - E))
    valp = jnp.pad(edge_vals, (0, EPADT - E))

    pre = _matvec(x, W0).reshape(NPAD)
    partials = _segsum(pre, dstp, srcp, valp)

    keys_i32 = _keys(partials.reshape(2, NPAD // 128, 128),
                     b.reshape(1, 1)).reshape(NPAD)

    topidx, topval = _sort_topk(keys_i32)

    out = _gather_scale(topidx, topval, x)
    return out[:K]
